# baseline (speedup 1.0000x reference)
import functools

import jax
import jax.numpy as jnp
from jax import lax
from jax.experimental import pallas as pl
from jax.experimental.pallas import tpu as pltpu
from jax.experimental.pallas import tpu_sc as plsc

K = 32
D = 64
N = 65536
B = 8
L = 16
C1 = 128
QUARTER = N // 4
NC1 = QUARTER // C1
C2 = 512
NC2 = QUARTER // C2
G2 = C2 // L
PUSH_MARGIN = 0.25
PUSH_WEIGHT = 1.0
PULL_WEIGHT = 1.0
REG_WEIGHT = 0.0001

_mesh = plsc.VectorSubcoreMesh(core_axis_name="c", subcore_axis_name="s")


@functools.partial(
    pl.kernel,
    out_type=(
        jax.ShapeDtypeStruct((B, L), jnp.float32),
        jax.ShapeDtypeStruct((32, L), jnp.float32),
    ),
    mesh=_mesh,
    compiler_params=pltpu.CompilerParams(needs_layout_passes=False),
    scratch_types=[
        pltpu.VMEM((C1, D), jnp.float32),
        pltpu.VMEM((C1, D), jnp.float32),
        pltpu.VMEM((C1,), jnp.int32),
        pltpu.VMEM((C1,), jnp.int32),
        pltpu.VMEM((C1,), jnp.int32),
        pltpu.VMEM((C1,), jnp.int32),
        pltpu.VMEM((C1, L), jnp.float32),
        pltpu.VMEM((C2 * D,), jnp.float32),
        pltpu.VMEM((C2 * D,), jnp.float32),
        pltpu.VMEM((C2,), jnp.int32),
        pltpu.VMEM((C2,), jnp.int32),
        pltpu.VMEM((K, D), jnp.float32),
        pltpu.VMEM((K * D,), jnp.float32),
        pltpu.VMEM((K, L), jnp.float32),
        pltpu.VMEM((L,), jnp.float32),
        pltpu.VMEM_SHARED((4 * K, D), jnp.float32),
        pltpu.VMEM_SHARED((4 * K, L), jnp.float32),
        pltpu.SemaphoreType.DMA,
        pltpu.SemaphoreType.DMA,
        pltpu.SemaphoreType.DMA,
        pltpu.SemaphoreType.DMA,
        pltpu.SemaphoreType.DMA,
        pltpu.SemaphoreType.DMA,
    ],
)
def _sc_loss(emb_hbm, embf_hbm, lab_hbm, pushreg_out, pull_out,
             e1a, e1b, l1a, l1b, ixa, ixb, ones_buf,
             e2a, e2b, l2a, l2b, cent2d, centf, cnt_buf, vec_buf,
             sums_sh, cnts_sh, se_a, se_b, sl_a, sl_b, sx_a, sx_b):
    c = lax.axis_index("c")
    s = lax.axis_index("s")
    local_s = s // 4
    quarter = s % 4
    sample = 4 * c + local_s
    wid = c * 16 + s
    base = quarter * QUARTER

    zeros16 = jnp.zeros((L,), jnp.float32)
    ones16 = jnp.ones((L,), jnp.float32)
    iota16 = lax.iota(jnp.int32, L)

    def _ones_body(r, _):
        ones_buf[r, :] = ones16
        return 0
    lax.fori_loop(0, C1, _ones_body, 0)

    @pl.when(s == 0)
    def _zero_shared():
        def _ze(r, _):
            for jj in range(D // L):
                e1a[r, pl.ds(L * jj, L)] = zeros16
            return 0
        lax.fori_loop(0, C1, _ze, 0)

        def _zc(r, _):
            cnt_buf[r, :] = zeros16
            return 0
        lax.fori_loop(0, K, _zc, 0)
        pltpu.sync_copy(e1a, sums_sh)
        for t in range(4):
            pltpu.sync_copy(cnt_buf, cnts_sh.at[pl.ds(t * K, K)])

    plsc.subcore_barrier()

    row_off = K * local_s

    def load1(k, eb, lb, se, sl):
        off = base + k * C1
        pltpu.async_copy(emb_hbm.at[sample, pl.ds(off, C1), :], eb, se)
        pltpu.async_copy(lab_hbm.at[sample, pl.ds(off, C1)], lb, sl)

    def wait1(eb, lb, se, sl):
        pltpu.make_async_copy(emb_hbm.at[sample, pl.ds(0, C1), :], eb, se).wait()
        pltpu.make_async_copy(lab_hbm.at[sample, pl.ds(0, C1)], lb, sl).wait()

    load1(0, e1a, l1a, se_a, sl_a)
    load1(1, e1b, l1b, se_b, sl_b)

    def p1_step(k, eb, lb, ix, se, sl, sx):
        wait1(eb, lb, se, sl)
        for g in range(C1 // L):
            lv = lb[pl.ds(L * g, L)]
            ix[pl.ds(L * g, L)] = lv + row_off
        pltpu.sync_copy(eb, sums_sh.at[ix], add=True)
        pltpu.sync_copy(ones_buf, cnts_sh.at[ix], add=True)
        kn = jnp.minimum(k + 2, NC1 - 1)
        load1(kn, eb, lb, se, sl)

    def _p1_body(p, _):
        p1_step(2 * p, e1a, l1a, ixa, se_a, sl_a, sx_a)
        p1_step(2 * p + 1, e1b, l1b, ixb, se_b, sl_b, sx_b)
        return 0

    lax.fori_loop(0, NC1 // 2, _p1_body, 0)
    wait1(e1a, l1a, se_a, sl_a)
    wait1(e1b, l1b, se_b, sl_b)
    plsc.subcore_barrier()

    pltpu.sync_copy(sums_sh.at[pl.ds(row_off, K)], cent2d)
    pltpu.sync_copy(cnts_sh.at[pl.ds(row_off, K)], cnt_buf)

    def _cent_body(r, _):
        cnt = cnt_buf[r, :]
        denom = jnp.maximum(cnt, 1.0)
        valid = cnt > 0.0
        for jj in range(D // L):
            sv = cent2d[r, pl.ds(L * jj, L)]
            centf[pl.ds(r * D + L * jj, L)] = jnp.where(valid, sv / denom, 0.0)
        return 0
    lax.fori_loop(0, K, _cent_body, 0)

    def _hsum_bcast(v):
        for m in (8, 4, 2, 1):
            vec_buf[:] = v
            v = v + plsc.load_gather(vec_buf, [iota16 ^ m])
        return v

    @pl.when(quarter == 0)
    def _push_reg():
        def _nv_body(r, acc):
            return acc + jnp.where(cnt_buf[r, :] > 0.0, ones16, zeros16)
        nv_vec = lax.fori_loop(0, K, _nv_body, zeros16)

        def _push_i(i, acc_i):
            ci = [centf[pl.ds(i * D + L * jj, L)] for jj in range(D // L)]
            vi = cnt_buf[i, :] > 0.0
            ivec = jnp.full((L,), i, jnp.int32)

            def _push_j(j, acc_j):
                dv = zeros16
                for jj in range(D // L):
                    dv = dv + jnp.abs(ci[jj] - centf[pl.ds(j * D + L * jj, L)])
                dist = _hsum_bcast(dv)
                vj = cnt_buf[j, :] > 0.0
                m = (ivec < jnp.full((L,), j, jnp.int32)) & vi & vj
                h = jnp.maximum(PUSH_MARGIN - dist, 0.0)
                return acc_j + jnp.where(m, h * h, zeros16)

            return lax.fori_loop(0, K, _push_j, acc_i)

        push_sum = lax.fori_loop(0, K, _push_i, zeros16)
        n_comp = nv_vec * (nv_vec - 1.0) * 0.5
        push_loss = jnp.where(nv_vec >= 2.0,
                              push_sum / jnp.maximum(n_comp, 1.0), zeros16)

        def _reg_body(r, acc):
            sq = zeros16
            for jj in range(D // L):
                cv = centf[pl.ds(r * D + L * jj, L)]
                sq = sq + cv * cv
            return acc + jnp.where(cnt_buf[r, :] > 0.0, sq, zeros16)
        reg_vec = _hsum_bcast(lax.fori_loop(0, K, _reg_body, zeros16))
        reg_loss = reg_vec / jnp.maximum(nv_vec * float(D), 1.0)

        vec_buf[:] = PUSH_WEIGHT * push_loss + REG_WEIGHT * reg_loss
        pltpu.sync_copy(vec_buf, pushreg_out.at[sample])

    def load2(k, eb, lb, se, sl):
        off = base + k * C2
        pltpu.async_copy(embf_hbm.at[sample, pl.ds(off * D, C2 * D)], eb, se)
        pltpu.async_copy(lab_hbm.at[sample, pl.ds(off, C2)], lb, sl)

    def wait2(eb, lb, se, sl):
        pltpu.make_async_copy(embf_hbm.at[sample, pl.ds(0, C2 * D)], eb, se).wait()
        pltpu.make_async_copy(lab_hbm.at[sample, pl.ds(0, C2)], lb, sl).wait()

    load2(0, e2a, l2a, se_a, sl_a)
    load2(1, e2b, l2b, se_b, sl_b)

    def p2_step(k, eb, lb, se, sl, pacc):
        wait2(eb, lb, se, sl)

        def g_body(g, acc):
            lv = lb[pl.ds(g * L, L)]
            pixb = (iota16 + g * L) * D
            labb = lv * D

            def d_body(dd, carry):
                a, ide, idc = carry
                for _ in range(4):
                    e = plsc.load_gather(eb, [ide])
                    cv = plsc.load_gather(centf, [idc])
                    a = a + jnp.abs(e - cv)
                    ide = ide + 1
                    idc = idc + 1
                return a, ide, idc

            pd, _, _ = lax.fori_loop(0, D // 4, d_body, (zeros16, pixb, labb))
            return acc + pd * pd

        pacc = lax.fori_loop(0, G2, g_body, pacc)
        kn = jnp.minimum(k + 2, NC2 - 1)
        load2(kn, eb, lb, se, sl)
        return pacc

    def _p2_body(p, pacc):
        pacc = p2_step(2 * p, e2a, l2a, se_a, sl_a, pacc)
        pacc = p2_step(2 * p + 1, e2b, l2b, se_b, sl_b, pacc)
        return pacc

    pacc = lax.fori_loop(0, NC2 // 2, _p2_body, zeros16)
    wait2(e2a, l2a, se_a, sl_a)
    wait2(e2b, l2b, se_b, sl_b)
    vec_buf[:] = pacc
    pltpu.sync_copy(vec_buf, pull_out.at[wid])


def kernel(embeddings, labels):
    lab32 = labels.astype(jnp.int32)
    embf = embeddings.reshape(B, N * D)
    pushreg, pull = _sc_loss(embeddings, embf, lab32)
    pull_s = pull.reshape(2, 4, 4 * L).sum(axis=-1).reshape(B)
    pull_loss = pull_s / float(N)
    return jnp.mean(pushreg[:, 0] + PULL_WEIGHT * pull_loss)
